# Initial kernel scaffold; baseline (speedup 1.0000x reference)
import jax
import jax.numpy as jnp
from jax.experimental import pallas as pl


def kernel(x, edge_index, batch, W1, b1, W2, b2, W3, b3):
    raise NotImplementedError("write your pallas kernel here")


# speedup vs baseline: 3.1002x; 3.1002x over previous
import functools

import jax
import jax.numpy as jnp
from jax import lax
from jax.experimental import pallas as pl
from jax.experimental.pallas import tpu as pltpu
from jax.experimental.pallas import tpu_sc as plsc
from jax.experimental.layout import Layout, with_layout_constraint

N = 100000
E = 1600000
H = 64
NUM_GRAPHS = 64
IN_DIM = 4

NC = 2
NS = 16
NW = NC * NS

N_PAD = 100352
ROWS_PT = N_PAD // NS
CH = 128
ECH = E // CH

R_BLK = 5000
NB = N // R_BLK
R2_BLK = 1000
NB2 = N // R2_BLK

_mesh = plsc.VectorSubcoreMesh(core_axis_name="c", subcore_axis_name="s")

_SC_ROW_LAYOUT2 = Layout(major_to_minor=(0, 1), tiling=((8,),))
_SC_ROW_LAYOUT3 = Layout(major_to_minor=(0, 1, 2), tiling=((8,),))


@functools.partial(
    pl.kernel,
    out_type=jax.ShapeDtypeStruct((NC, N_PAD), jnp.float32),
    mesh=_mesh,
    scratch_types=[
        pltpu.VMEM((CH,), jnp.int32),
        pltpu.VMEM((CH,), jnp.float32),
        pltpu.VMEM_SHARED((N_PAD,), jnp.float32),
    ],
)
def _sc_deg(dst_hbm, zeros_hbm, out_hbm, idx_v, ones_v, accum):
    c = lax.axis_index("c")
    s = lax.axis_index("s")

    @pl.loop(0, CH, step=16)
    def _fill(i):
        ones_v[pl.ds(i, 16)] = jnp.ones((16,), jnp.float32)

    pltpu.sync_copy(zeros_hbm.at[pl.ds(s * ROWS_PT, ROWS_PT)],
                    accum.at[pl.ds(s * ROWS_PT, ROWS_PT)])
    plsc.subcore_barrier()

    w = c * NS + s
    cbase = w * (ECH // NW) + jnp.minimum(w, ECH % NW)
    nchunks = ECH // NW + jnp.where(w < ECH % NW, 1, 0)

    @pl.loop(0, nchunks)
    def _edges(k):
        pltpu.sync_copy(dst_hbm.at[pl.ds((cbase + k) * CH, CH)], idx_v)
        pltpu.sync_copy(ones_v, accum.at[idx_v], add=True)

    plsc.subcore_barrier()
    pltpu.sync_copy(accum.at[pl.ds(s * ROWS_PT, ROWS_PT)],
                    out_hbm.at[c, pl.ds(s * ROWS_PT, ROWS_PT)])


def _stage1_body(x_ref, a_ref, dinv_ref, w1_ref, b1_ref, out_ref):
    dinv = dinv_ref[...]
    z = (a_ref[...] + x_ref[...] * dinv) * dinv
    h = jnp.dot(z, w1_ref[...], preferred_element_type=jnp.float32) + b1_ref[...]
    out_ref[...] = jnp.maximum(h, 0.0) * dinv


def _tc_stage1(x, agg1, dinv2d, W1, b1r):
    return pl.pallas_call(
        _stage1_body,
        grid=(NB,),
        in_specs=[
            pl.BlockSpec((R_BLK, IN_DIM), lambda bi: (bi, 0)),
            pl.BlockSpec((R_BLK, IN_DIM), lambda bi: (bi, 0)),
            pl.BlockSpec((R_BLK, 1), lambda bi: (bi, 0)),
            pl.BlockSpec((IN_DIM, H), lambda bi: (0, 0)),
            pl.BlockSpec((1, H), lambda bi: (0, 0)),
        ],
        out_specs=pl.BlockSpec((R_BLK, H), lambda bi: (bi, 0)),
        out_shape=jax.ShapeDtypeStruct((N, H), jnp.float32),
    )(x, agg1, dinv2d, W1, b1r)


def _stage2_body(g_ref, a_ref, dinv_ref, batch_ref,
                 w2_ref, b2_ref, w3_ref, sums_ref, counts_ref):
    i = pl.program_id(0)
    z2 = (a_ref[...] + g_ref[...]) * dinv_ref[...]
    h2 = jnp.maximum(
        jnp.dot(z2, w2_ref[...], preferred_element_type=jnp.float32)
        + b2_ref[...], 0.0)
    sv = jnp.dot(h2, w3_ref[...], preferred_element_type=jnp.float32)
    onehot = (batch_ref[...] == lax.broadcasted_iota(
        jnp.int32, (1, NUM_GRAPHS), 1)).astype(jnp.float32)
    part_s = lax.dot_general(sv, onehot, (((0,), (0,)), ((), ())))
    part_c = jnp.sum(onehot, axis=0, keepdims=True)

    @pl.when(i == 0)
    def _():
        sums_ref[...] = jnp.zeros_like(sums_ref)
        counts_ref[...] = jnp.zeros_like(counts_ref)

    sums_ref[...] += part_s
    counts_ref[...] += part_c


def _tc_stage2(g, agg2, dinv2d, batch2d, W2, b2r, W3):
    return pl.pallas_call(
        _stage2_body,
        grid=(NB2,),
        in_specs=[pl.BlockSpec((R2_BLK, H), lambda i: (i, 0)),
                  pl.BlockSpec((R2_BLK, H), lambda i: (i, 0)),
                  pl.BlockSpec((R2_BLK, 1), lambda i: (i, 0)),
                  pl.BlockSpec((R2_BLK, 1), lambda i: (i, 0)),
                  pl.BlockSpec((H, H), lambda i: (0, 0)),
                  pl.BlockSpec((1, H), lambda i: (0, 0)),
                  pl.BlockSpec((H, 1), lambda i: (0, 0))],
        out_specs=[pl.BlockSpec((1, NUM_GRAPHS), lambda i: (0, 0)),
                   pl.BlockSpec((1, NUM_GRAPHS), lambda i: (0, 0))],
        out_shape=[jax.ShapeDtypeStruct((1, NUM_GRAPHS), jnp.float32),
                   jax.ShapeDtypeStruct((1, NUM_GRAPHS), jnp.float32)],
    )(g, agg2, dinv2d, batch2d, W2, b2r, W3)


def kernel(x, edge_index, batch, W1, b1, W2, b2, W3, b3):
    src = edge_index[0]
    dst = edge_index[1]

    zeros1 = jnp.zeros((N_PAD,), jnp.float32)
    degp = _sc_deg(dst, zeros1)
    deg = degp[0, :N] + degp[1, :N] + 1.0
    dinv = lax.rsqrt(deg)
    dinv2d = dinv[:, None]

    xs = x * dinv2d
    agg1 = jnp.zeros((N, IN_DIM), jnp.float32).at[dst].add(xs[src])
    g = _tc_stage1(x, agg1, dinv2d, W1, b1[None, :])
    agg2 = jnp.zeros((N, H), jnp.float32).at[dst].add(g[src])
    sums, counts = _tc_stage2(g, agg2, dinv2d, batch[:, None], W2,
                              b2[None, :], W3)
    out = (sums / jnp.maximum(counts, 1.0)).reshape(NUM_GRAPHS, 1) + b3
    return out
